# baseline (speedup 1.0000x reference)
import functools

import jax
import jax.numpy as jnp
from jax import lax
from jax.experimental import pallas as pl
from jax.experimental.pallas import tpu as pltpu
from jax.experimental.pallas import tpu_sc as plsc

N = 10000
D = 256
DH = 128
E = 160000
NC, NS = 2, 16
K = 125
ROWS = E // K
ROWS_T_HIST = ROWS // (NC * NS)
KS = 100
RT = E // KS // NS
NPHASE = 4
PH = RT // NPHASE
RB = 1000

_mesh = plsc.VectorSubcoreMesh(
    core_axis_name="c", subcore_axis_name="s", num_cores=NC, num_subcores=NS
)


@functools.partial(
    pl.kernel,
    mesh=_mesh,
    out_type=[
        jax.ShapeDtypeStruct((N,), jnp.float32),
        jax.ShapeDtypeStruct((N,), jnp.float32),
    ],
    scratch_types=[
        pltpu.VMEM((ROWS_T_HIST, K), jnp.int32),
        pltpu.VMEM((128,), jnp.float32),
        pltpu.VMEM((640,), jnp.float32),
        pltpu.VMEM_SHARED((N,), jnp.float32),
    ],
)
def _sc_hist(dst2d_hbm, deg0_hbm, deg1_hbm, idx_v, ones_v, zeros_v, deg_sh):
    c = lax.axis_index("c")
    s = lax.axis_index("s")
    one16 = jnp.full((16,), 1.0, jnp.float32)
    zero16 = jnp.zeros((16,), jnp.float32)
    for i in range(8):
        ones_v[pl.ds(i * 16, 16)] = one16
    for i in range(40):
        zeros_v[pl.ds(i * 16, 16)] = zero16

    @pl.when(s < 15)
    def _():
        pltpu.sync_copy(zeros_v, deg_sh.at[pl.ds(s * 640, 640)])

    @pl.when(s == 15)
    def _():
        pltpu.sync_copy(zeros_v.at[pl.ds(0, 400)], deg_sh.at[pl.ds(9600, 400)])

    plsc.subcore_barrier()

    wid = c * NS + s
    pltpu.sync_copy(dst2d_hbm.at[pl.ds(wid * ROWS_T_HIST, ROWS_T_HIST)], idx_v)
    for j in range(ROWS_T_HIST):
        pltpu.sync_copy(ones_v.at[pl.ds(0, K)], deg_sh.at[idx_v.at[j]], add=True)

    plsc.subcore_barrier()

    def _write(out_hbm):
        @pl.when(s < 15)
        def _():
            pltpu.sync_copy(deg_sh.at[pl.ds(s * 640, 640)], zeros_v)
            pltpu.sync_copy(zeros_v, out_hbm.at[pl.ds(s * 640, 640)])

        @pl.when(s == 15)
        def _():
            pltpu.sync_copy(deg_sh.at[pl.ds(9600, 400)],
                            zeros_v.at[pl.ds(0, 400)])
            pltpu.sync_copy(zeros_v.at[pl.ds(0, 400)],
                            out_hbm.at[pl.ds(9600, 400)])

    @pl.when(c == 0)
    def _():
        _write(deg0_hbm)

    @pl.when(c == 1)
    def _():
        _write(deg1_hbm)


@functools.partial(
    pl.kernel,
    mesh=_mesh,
    out_type=[
        jax.ShapeDtypeStruct((N, DH), jnp.float32),
        jax.ShapeDtypeStruct((N, DH), jnp.float32),
    ],
    scratch_types=[
        pltpu.VMEM((PH, KS), jnp.int32),
        pltpu.VMEM((PH, KS), jnp.int32),
        pltpu.VMEM((3, KS, DH), jnp.float32),
        pltpu.SemaphoreType.DMA,
        pltpu.SemaphoreType.DMA,
        pltpu.VMEM_SHARED((N, DH), jnp.float32),
    ],
)
def _sc_scatter(ga_hbm, gb_hbm, src3d_hbm, dst3d_hbm, outa_hbm, outb_hbm,
                src_v, dst_v, rows_v, gsem, ssem, acc_sh):
    c = lax.axis_index("c")
    s = lax.axis_index("s")
    zero16 = jnp.zeros((16,), jnp.float32)
    buf0 = rows_v.at[0]

    def _zrow(i, carry):
        for j in range(DH // 16):
            rows_v[0, i, pl.ds(j * 16, 16)] = zero16
        return carry

    lax.fori_loop(0, KS, _zrow, 0)

    def _span(fn, total):
        off = 0
        while off + 80 <= total:
            fn(off, 80)
            off += 80
        if off < total:
            fn(off, total - off)

    @pl.when(s < 15)
    def _():
        _span(lambda o, n: pltpu.sync_copy(
            buf0.at[pl.ds(0, n)], acc_sh.at[pl.ds(s * 632 + o, n)]), 632)

    @pl.when(s == 15)
    def _():
        _span(lambda o, n: pltpu.sync_copy(
            buf0.at[pl.ds(0, n)], acc_sh.at[pl.ds(9480 + o, n)]), 520)

    plsc.subcore_barrier()

    def _run(g_hbm):
        def _gather(j, b):
            idx = src_v.at[j]
            buf = rows_v.at[b]
            pltpu.async_copy(g_hbm.at[idx.at[pl.ds(0, 48)]],
                             buf.at[pl.ds(0, 48)], gsem)
            pltpu.async_copy(g_hbm.at[idx.at[pl.ds(48, 52)]],
                             buf.at[pl.ds(48, 52)], gsem)

        def _wait_gather():
            pltpu.make_async_copy(g_hbm.at[src_v.at[0]], buf0, gsem).wait()

        def _wait_one(sem):
            pltpu.make_async_copy(g_hbm.at[src_v.at[0]], buf0, sem).wait()

        for phase in range(NPHASE):
            pltpu.sync_copy(src3d_hbm.at[s * NPHASE + phase], src_v)
            pltpu.sync_copy(dst3d_hbm.at[s * NPHASE + phase], dst_v)
            _gather(0, 0)
            _gather(1, 1)

            def _chunk(j, carry):
                b = lax.rem(j, 3)
                buf = rows_v.at[b]
                _wait_gather()
                pltpu.async_copy(buf, acc_sh.at[dst_v.at[j]], ssem, add=True)

                @pl.when(j + 2 < PH)
                def _():
                    @pl.when(j >= 1)
                    def _():
                        _wait_one(ssem)
                    _gather(j + 2, lax.rem(j + 2, 3))

                return carry

            lax.fori_loop(0, PH, _chunk, 0)
            for _ in range(3):
                _wait_one(ssem)

    @pl.when(c == 0)
    def _():
        _run(ga_hbm)

    @pl.when(c == 1)
    def _():
        _run(gb_hbm)

    plsc.subcore_barrier()

    def _bounce(out_hbm, base, size):
        pltpu.sync_copy(acc_sh.at[pl.ds(base, size)],
                        buf0.at[pl.ds(0, size)])
        pltpu.sync_copy(buf0.at[pl.ds(0, size)],
                        out_hbm.at[pl.ds(base, size)])

    def _write(out_hbm):
        @pl.when(s < 15)
        def _():
            _span(lambda o, n: _bounce(out_hbm, s * 632 + o, n), 632)

        @pl.when(s == 15)
        def _():
            _span(lambda o, n: _bounce(out_hbm, 9480 + o, n), 520)

    @pl.when(c == 0)
    def _():
        _write(outa_hbm)

    @pl.when(c == 1)
    def _():
        _write(outb_hbm)


def _dinv(d0_ref, d1_ref):
    return lax.rsqrt(1.0 + d0_ref[...] + d1_ref[...])


def _tc1_body(x_ref, w_ref, d0_ref, d1_ref, ga_ref, gb_ref):
    h = jnp.dot(x_ref[...], w_ref[...],
                preferred_element_type=jnp.float32,
                precision=lax.Precision.HIGHEST)
    g = h * _dinv(d0_ref, d1_ref)
    ga_ref[...] = g[:, :DH]
    gb_ref[...] = g[:, DH:]


def _tc2_body(sa_ref, sb_ref, ga_ref, gb_ref, d0_ref, d1_ref, w_ref,
              g2a_ref, g2b_ref):
    dinv = _dinv(d0_ref, d1_ref)
    xa = jnp.maximum(dinv * (sa_ref[...] + ga_ref[...]), 0.0)
    xb = jnp.maximum(dinv * (sb_ref[...] + gb_ref[...]), 0.0)
    x2 = jnp.concatenate([xa, xb], axis=1)
    t = jnp.dot(x2, w_ref[...],
                preferred_element_type=jnp.float32,
                precision=lax.Precision.HIGHEST)
    g2 = t * dinv
    g2a_ref[...] = g2[:, :DH]
    g2b_ref[...] = g2[:, DH:]


def _tc3_body(sa_ref, sb_ref, ga_ref, gb_ref, d0_ref, d1_ref, out_ref):
    dinv = _dinv(d0_ref, d1_ref)
    oa = jnp.maximum(dinv * (sa_ref[...] + ga_ref[...]), 0.0)
    ob = jnp.maximum(dinv * (sb_ref[...] + gb_ref[...]), 0.0)
    out_ref[...] = jnp.concatenate([oa, ob], axis=1)


_row_half = pl.BlockSpec((RB, DH), lambda i: (i, 0))
_row_full = pl.BlockSpec((RB, D), lambda i: (i, 0))
_col_deg = pl.BlockSpec((RB, 1), lambda i: (i, 0))
_w_full = pl.BlockSpec((D, D), lambda i: (0, 0))
_half_out = jax.ShapeDtypeStruct((N, DH), jnp.float32)

_tc1 = pl.pallas_call(
    _tc1_body,
    grid=(N // RB,),
    in_specs=[_row_full, _w_full, _col_deg, _col_deg],
    out_specs=[_row_half, _row_half],
    out_shape=[_half_out, _half_out],
)

_tc2 = pl.pallas_call(
    _tc2_body,
    grid=(N // RB,),
    in_specs=[_row_half, _row_half, _row_half, _row_half,
              _col_deg, _col_deg, _w_full],
    out_specs=[_row_half, _row_half],
    out_shape=[_half_out, _half_out],
)

_tc3 = pl.pallas_call(
    _tc3_body,
    grid=(N // RB,),
    in_specs=[_row_half, _row_half, _row_half, _row_half,
              _col_deg, _col_deg],
    out_specs=_row_full,
    out_shape=jax.ShapeDtypeStruct((N, D), jnp.float32),
)


def kernel(x, edge_index, W1, W2):
    src = edge_index[0].astype(jnp.int32)
    dst = edge_index[1].astype(jnp.int32)
    dst2d = dst.reshape(ROWS, K)
    src3d = src.reshape(NS * NPHASE, PH, KS)
    dst3d = dst.reshape(NS * NPHASE, PH, KS)

    deg0, deg1 = _sc_hist(dst2d)
    d0 = deg0.reshape(N, 1)
    d1 = deg1.reshape(N, 1)

    ga, gb = _tc1(x, W1, d0, d1)
    sa, sb = _sc_scatter(ga, gb, src3d, dst3d)
    g2a, g2b = _tc2(sa, sb, ga, gb, d0, d1, W2)
    s2a, s2b = _sc_scatter(g2a, g2b, src3d, dst3d)
    return _tc3(s2a, s2b, g2a, g2b, d0, d1)


# speedup vs baseline: 1.0025x; 1.0025x over previous
import functools

import jax
import jax.numpy as jnp
from jax import lax
from jax.experimental import pallas as pl
from jax.experimental.pallas import tpu as pltpu
from jax.experimental.pallas import tpu_sc as plsc

N = 10000
D = 256
DH = 128
E = 160000
NC, NS = 2, 16
K = 125
ROWS = E // K
ROWS_T_HIST = ROWS // (NC * NS)
KS = 100
RT = E // KS // NS
NPHASE = 4
PH = RT // NPHASE
RB = 1000

_mesh = plsc.VectorSubcoreMesh(
    core_axis_name="c", subcore_axis_name="s", num_cores=NC, num_subcores=NS
)


@functools.partial(
    pl.kernel,
    mesh=_mesh,
    out_type=[
        jax.ShapeDtypeStruct((N,), jnp.float32),
        jax.ShapeDtypeStruct((N,), jnp.float32),
    ],
    scratch_types=[
        pltpu.VMEM((ROWS_T_HIST, K), jnp.int32),
        pltpu.VMEM((128,), jnp.float32),
        pltpu.VMEM((640,), jnp.float32),
        pltpu.VMEM_SHARED((N,), jnp.float32),
    ],
)
def _sc_hist(dst2d_hbm, deg0_hbm, deg1_hbm, idx_v, ones_v, zeros_v, deg_sh):
    c = lax.axis_index("c")
    s = lax.axis_index("s")
    one16 = jnp.full((16,), 1.0, jnp.float32)
    zero16 = jnp.zeros((16,), jnp.float32)
    for i in range(8):
        ones_v[pl.ds(i * 16, 16)] = one16
    for i in range(40):
        zeros_v[pl.ds(i * 16, 16)] = zero16

    @pl.when(s < 15)
    def _():
        pltpu.sync_copy(zeros_v, deg_sh.at[pl.ds(s * 640, 640)])

    @pl.when(s == 15)
    def _():
        pltpu.sync_copy(zeros_v.at[pl.ds(0, 400)], deg_sh.at[pl.ds(9600, 400)])

    plsc.subcore_barrier()

    wid = c * NS + s
    pltpu.sync_copy(dst2d_hbm.at[pl.ds(wid * ROWS_T_HIST, ROWS_T_HIST)], idx_v)
    for j in range(ROWS_T_HIST):
        pltpu.sync_copy(ones_v.at[pl.ds(0, K)], deg_sh.at[idx_v.at[j]], add=True)

    plsc.subcore_barrier()

    def _write(out_hbm):
        @pl.when(s < 15)
        def _():
            pltpu.sync_copy(deg_sh.at[pl.ds(s * 640, 640)], zeros_v)
            pltpu.sync_copy(zeros_v, out_hbm.at[pl.ds(s * 640, 640)])

        @pl.when(s == 15)
        def _():
            pltpu.sync_copy(deg_sh.at[pl.ds(9600, 400)],
                            zeros_v.at[pl.ds(0, 400)])
            pltpu.sync_copy(zeros_v.at[pl.ds(0, 400)],
                            out_hbm.at[pl.ds(9600, 400)])

    @pl.when(c == 0)
    def _():
        _write(deg0_hbm)

    @pl.when(c == 1)
    def _():
        _write(deg1_hbm)


@functools.partial(
    pl.kernel,
    mesh=_mesh,
    out_type=[
        jax.ShapeDtypeStruct((N, DH), jnp.float32),
        jax.ShapeDtypeStruct((N, DH), jnp.float32),
    ],
    scratch_types=[
        pltpu.VMEM((PH, KS), jnp.int32),
        pltpu.VMEM((PH, KS), jnp.int32),
        pltpu.VMEM((3, KS, DH), jnp.float32),
        pltpu.SemaphoreType.DMA,
        pltpu.SemaphoreType.DMA,
        pltpu.VMEM_SHARED((N, DH), jnp.float32),
    ],
)
def _sc_scatter(ga_hbm, gb_hbm, src3d_hbm, dst3d_hbm, outa_hbm, outb_hbm,
                src_v, dst_v, rows_v, gsem, ssem, acc_sh):
    c = lax.axis_index("c")
    s = lax.axis_index("s")
    zero16 = jnp.zeros((16,), jnp.float32)
    buf0 = rows_v.at[0]

    def _zrow(i, carry):
        for j in range(DH // 16):
            rows_v[0, i, pl.ds(j * 16, 16)] = zero16
        return carry

    lax.fori_loop(0, KS, _zrow, 0)

    def _span(fn, total):
        off = 0
        while off + 80 <= total:
            fn(off, 80)
            off += 80
        if off < total:
            fn(off, total - off)

    @pl.when(s < 15)
    def _():
        _span(lambda o, n: pltpu.sync_copy(
            buf0.at[pl.ds(0, n)], acc_sh.at[pl.ds(s * 632 + o, n)]), 632)

    @pl.when(s == 15)
    def _():
        _span(lambda o, n: pltpu.sync_copy(
            buf0.at[pl.ds(0, n)], acc_sh.at[pl.ds(9480 + o, n)]), 520)

    plsc.subcore_barrier()

    def _run(g_hbm):
        def _gather(j, b):
            pltpu.async_copy(g_hbm.at[src_v.at[j]], rows_v.at[b], gsem)

        def _wait_gather():
            pltpu.make_async_copy(g_hbm.at[src_v.at[0]], buf0, gsem).wait()

        def _wait_one(sem):
            pltpu.make_async_copy(g_hbm.at[src_v.at[0]], buf0, sem).wait()

        for phase in range(NPHASE):
            pltpu.sync_copy(src3d_hbm.at[s * NPHASE + phase], src_v)
            pltpu.sync_copy(dst3d_hbm.at[s * NPHASE + phase], dst_v)
            _gather(0, 0)
            _gather(1, 1)

            def _chunk(j, carry):
                b = lax.rem(j, 3)
                buf = rows_v.at[b]
                _wait_gather()
                pltpu.async_copy(buf, acc_sh.at[dst_v.at[j]], ssem, add=True)

                @pl.when(j + 2 < PH)
                def _():
                    @pl.when(j >= 1)
                    def _():
                        _wait_one(ssem)
                    _gather(j + 2, lax.rem(j + 2, 3))

                return carry

            lax.fori_loop(0, PH, _chunk, 0)
            for _ in range(3):
                _wait_one(ssem)

    @pl.when(c == 0)
    def _():
        _run(ga_hbm)

    @pl.when(c == 1)
    def _():
        _run(gb_hbm)

    plsc.subcore_barrier()

    def _bounce(out_hbm, base, size):
        pltpu.sync_copy(acc_sh.at[pl.ds(base, size)],
                        buf0.at[pl.ds(0, size)])
        pltpu.sync_copy(buf0.at[pl.ds(0, size)],
                        out_hbm.at[pl.ds(base, size)])

    def _write(out_hbm):
        @pl.when(s < 15)
        def _():
            _span(lambda o, n: _bounce(out_hbm, s * 632 + o, n), 632)

        @pl.when(s == 15)
        def _():
            _span(lambda o, n: _bounce(out_hbm, 9480 + o, n), 520)

    @pl.when(c == 0)
    def _():
        _write(outa_hbm)

    @pl.when(c == 1)
    def _():
        _write(outb_hbm)


def _dinv(d0_ref, d1_ref):
    return lax.rsqrt(1.0 + d0_ref[...] + d1_ref[...])


def _tc1a_body(x_ref, w_ref, h_ref):
    h_ref[...] = jnp.dot(x_ref[...], w_ref[...],
                         preferred_element_type=jnp.float32)


def _tc1b_body(h_ref, d0_ref, d1_ref, ga_ref, gb_ref):
    g = h_ref[...] * _dinv(d0_ref, d1_ref)
    ga_ref[...] = g[:, :DH]
    gb_ref[...] = g[:, DH:]


def _tc2_body(sa_ref, sb_ref, ga_ref, gb_ref, d0_ref, d1_ref, w_ref,
              g2a_ref, g2b_ref):
    dinv = _dinv(d0_ref, d1_ref)
    xa = jnp.maximum(dinv * (sa_ref[...] + ga_ref[...]), 0.0)
    xb = jnp.maximum(dinv * (sb_ref[...] + gb_ref[...]), 0.0)
    x2 = jnp.concatenate([xa, xb], axis=1)
    t = jnp.dot(x2, w_ref[...],
                preferred_element_type=jnp.float32)
    g2 = t * dinv
    g2a_ref[...] = g2[:, :DH]
    g2b_ref[...] = g2[:, DH:]


def _tc3_body(sa_ref, sb_ref, ga_ref, gb_ref, d0_ref, d1_ref, out_ref):
    dinv = _dinv(d0_ref, d1_ref)
    oa = jnp.maximum(dinv * (sa_ref[...] + ga_ref[...]), 0.0)
    ob = jnp.maximum(dinv * (sb_ref[...] + gb_ref[...]), 0.0)
    out_ref[...] = jnp.concatenate([oa, ob], axis=1)


_row_half = pl.BlockSpec((RB, DH), lambda i: (i, 0))
_row_full = pl.BlockSpec((RB, D), lambda i: (i, 0))
_col_deg = pl.BlockSpec((RB, 1), lambda i: (i, 0))
_w_full = pl.BlockSpec((D, D), lambda i: (0, 0))
_half_out = jax.ShapeDtypeStruct((N, DH), jnp.float32)

_tc1a = pl.pallas_call(
    _tc1a_body,
    grid=(N // RB,),
    in_specs=[_row_full, _w_full],
    out_specs=_row_full,
    out_shape=jax.ShapeDtypeStruct((N, D), jnp.float32),
)

_tc1b = pl.pallas_call(
    _tc1b_body,
    grid=(N // RB,),
    in_specs=[_row_full, _col_deg, _col_deg],
    out_specs=[_row_half, _row_half],
    out_shape=[_half_out, _half_out],
)

_tc2 = pl.pallas_call(
    _tc2_body,
    grid=(N // RB,),
    in_specs=[_row_half, _row_half, _row_half, _row_half,
              _col_deg, _col_deg, _w_full],
    out_specs=[_row_half, _row_half],
    out_shape=[_half_out, _half_out],
)

_tc3 = pl.pallas_call(
    _tc3_body,
    grid=(N // RB,),
    in_specs=[_row_half, _row_half, _row_half, _row_half,
              _col_deg, _col_deg],
    out_specs=_row_full,
    out_shape=jax.ShapeDtypeStruct((N, D), jnp.float32),
)


def kernel(x, edge_index, W1, W2):
    src = edge_index[0].astype(jnp.int32)
    dst = edge_index[1].astype(jnp.int32)
    dst2d = dst.reshape(ROWS, K)
    src3d = src.reshape(NS * NPHASE, PH, KS)
    dst3d = dst.reshape(NS * NPHASE, PH, KS)

    h1 = _tc1a(x, W1)
    deg0, deg1 = _sc_hist(dst2d)
    d0 = deg0.reshape(N, 1)
    d1 = deg1.reshape(N, 1)

    ga, gb = _tc1b(h1, d0, d1)
    sa, sb = _sc_scatter(ga, gb, src3d, dst3d)
    g2a, g2b = _tc2(sa, sb, ga, gb, d0, d1, W2)
    s2a, s2b = _sc_scatter(g2a, g2b, src3d, dst3d)
    return _tc3(s2a, s2b, g2a, g2b, d0, d1)


# speedup vs baseline: 1.0160x; 1.0135x over previous
import functools

import jax
import jax.numpy as jnp
from jax import lax
from jax.experimental import pallas as pl
from jax.experimental.pallas import tpu as pltpu
from jax.experimental.pallas import tpu_sc as plsc

N = 10000
D = 256
DH = 128
E = 160000
NC, NS = 2, 16
K = 125
ROWS = E // K
ROWS_T_HIST = ROWS // (NC * NS)
KS = 100
RT = E // KS // NS
NPHASE = 4
PH = RT // NPHASE
RB = 1000

_mesh = plsc.VectorSubcoreMesh(
    core_axis_name="c", subcore_axis_name="s", num_cores=NC, num_subcores=NS
)


@functools.partial(
    pl.kernel,
    mesh=_mesh,
    out_type=[
        jax.ShapeDtypeStruct((N,), jnp.float32),
        jax.ShapeDtypeStruct((N,), jnp.float32),
    ],
    scratch_types=[
        pltpu.VMEM((ROWS_T_HIST, K), jnp.int32),
        pltpu.VMEM((128,), jnp.float32),
        pltpu.VMEM((640,), jnp.float32),
        pltpu.VMEM_SHARED((N,), jnp.float32),
    ],
)
def _sc_hist(dst2d_hbm, deg0_hbm, deg1_hbm, idx_v, ones_v, zeros_v, deg_sh):
    c = lax.axis_index("c")
    s = lax.axis_index("s")
    one16 = jnp.full((16,), 1.0, jnp.float32)
    zero16 = jnp.zeros((16,), jnp.float32)
    for i in range(8):
        ones_v[pl.ds(i * 16, 16)] = one16
    for i in range(40):
        zeros_v[pl.ds(i * 16, 16)] = zero16

    @pl.when(s < 15)
    def _():
        pltpu.sync_copy(zeros_v, deg_sh.at[pl.ds(s * 640, 640)])

    @pl.when(s == 15)
    def _():
        pltpu.sync_copy(zeros_v.at[pl.ds(0, 400)], deg_sh.at[pl.ds(9600, 400)])

    plsc.subcore_barrier()

    wid = c * NS + s
    pltpu.sync_copy(dst2d_hbm.at[pl.ds(wid * ROWS_T_HIST, ROWS_T_HIST)], idx_v)
    for j in range(ROWS_T_HIST):
        pltpu.sync_copy(ones_v.at[pl.ds(0, K)], deg_sh.at[idx_v.at[j]], add=True)

    plsc.subcore_barrier()

    def _write(out_hbm):
        @pl.when(s < 15)
        def _():
            pltpu.sync_copy(deg_sh.at[pl.ds(s * 640, 640)], zeros_v)
            pltpu.sync_copy(zeros_v, out_hbm.at[pl.ds(s * 640, 640)])

        @pl.when(s == 15)
        def _():
            pltpu.sync_copy(deg_sh.at[pl.ds(9600, 400)],
                            zeros_v.at[pl.ds(0, 400)])
            pltpu.sync_copy(zeros_v.at[pl.ds(0, 400)],
                            out_hbm.at[pl.ds(9600, 400)])

    @pl.when(c == 0)
    def _():
        _write(deg0_hbm)

    @pl.when(c == 1)
    def _():
        _write(deg1_hbm)


@functools.partial(
    pl.kernel,
    mesh=_mesh,
    out_type=[
        jax.ShapeDtypeStruct((N, DH), jnp.float32),
        jax.ShapeDtypeStruct((N, DH), jnp.float32),
    ],
    scratch_types=[
        pltpu.VMEM((PH, KS), jnp.int32),
        pltpu.VMEM((PH, KS), jnp.int32),
        pltpu.VMEM((3, KS, DH), jnp.float32),
        pltpu.SemaphoreType.DMA,
        pltpu.SemaphoreType.DMA,
        pltpu.VMEM_SHARED((N, DH), jnp.float32),
    ],
)
def _sc_scatter(ga_hbm, gb_hbm, src3d_hbm, dst3d_hbm, outa_hbm, outb_hbm,
                src_v, dst_v, rows_v, gsem, ssem, acc_sh):
    c = lax.axis_index("c")
    s = lax.axis_index("s")
    zero16 = jnp.zeros((16,), jnp.float32)
    buf0 = rows_v.at[0]

    def _zrow(i, carry):
        for j in range(DH // 16):
            rows_v[0, i, pl.ds(j * 16, 16)] = zero16
        return carry

    lax.fori_loop(0, KS, _zrow, 0)

    def _span(fn, total):
        off = 0
        while off + 80 <= total:
            fn(off, 80)
            off += 80
        if off < total:
            fn(off, total - off)

    def _zero(base, total):
        _span(lambda o, n: pltpu.async_copy(
            buf0.at[pl.ds(0, n)], acc_sh.at[pl.ds(base + o, n)], ssem), total)

    def _drain_zero(base, total):
        _span(lambda o, n: pltpu.make_async_copy(
            buf0.at[pl.ds(0, n)], acc_sh.at[pl.ds(base + o, n)],
            ssem).wait(), total)

    @pl.when(s < 15)
    def _():
        _zero(s * 632, 632)

    @pl.when(s == 15)
    def _():
        _zero(9480, 520)

    pltpu.sync_copy(src3d_hbm.at[s * NPHASE], src_v)
    pltpu.sync_copy(dst3d_hbm.at[s * NPHASE], dst_v)

    @pl.when(s < 15)
    def _():
        _drain_zero(s * 632, 632)

    @pl.when(s == 15)
    def _():
        _drain_zero(9480, 520)

    plsc.subcore_barrier()

    def _run(g_hbm):
        def _gather(j, b):
            pltpu.async_copy(g_hbm.at[src_v.at[j]], rows_v.at[b], gsem)

        def _wait_gather():
            pltpu.make_async_copy(g_hbm.at[src_v.at[0]], buf0, gsem).wait()

        def _wait_one(sem):
            pltpu.make_async_copy(g_hbm.at[src_v.at[0]], buf0, sem).wait()

        for phase in range(NPHASE):
            if phase > 0:
                pltpu.sync_copy(src3d_hbm.at[s * NPHASE + phase], src_v)
                pltpu.sync_copy(dst3d_hbm.at[s * NPHASE + phase], dst_v)
            _gather(0, 0)
            _gather(1, 1)

            def _chunk(j, carry):
                b = lax.rem(j, 3)
                buf = rows_v.at[b]
                _wait_gather()
                pltpu.async_copy(buf, acc_sh.at[dst_v.at[j]], ssem, add=True)

                @pl.when(j + 2 < PH)
                def _():
                    @pl.when(j >= 1)
                    def _():
                        _wait_one(ssem)
                    _gather(j + 2, lax.rem(j + 2, 3))

                return carry

            lax.fori_loop(0, PH, _chunk, 0)
            for _ in range(3):
                _wait_one(ssem)

    @pl.when(c == 0)
    def _():
        _run(ga_hbm)

    @pl.when(c == 1)
    def _():
        _run(gb_hbm)

    plsc.subcore_barrier()

    def _bounce(out_hbm, base, size):
        pltpu.sync_copy(acc_sh.at[pl.ds(base, size)],
                        buf0.at[pl.ds(0, size)])
        pltpu.sync_copy(buf0.at[pl.ds(0, size)],
                        out_hbm.at[pl.ds(base, size)])

    def _write(out_hbm):
        @pl.when(s < 15)
        def _():
            _span(lambda o, n: _bounce(out_hbm, s * 632 + o, n), 632)

        @pl.when(s == 15)
        def _():
            _span(lambda o, n: _bounce(out_hbm, 9480 + o, n), 520)

    @pl.when(c == 0)
    def _():
        _write(outa_hbm)

    @pl.when(c == 1)
    def _():
        _write(outb_hbm)


def _dinv(d0_ref, d1_ref):
    return lax.rsqrt(1.0 + d0_ref[...] + d1_ref[...])


def _tc1a_body(x_ref, w_ref, h_ref):
    h_ref[...] = jnp.dot(x_ref[...], w_ref[...],
                         preferred_element_type=jnp.float32)


def _tc1b_body(h_ref, d0_ref, d1_ref, ga_ref, gb_ref):
    g = h_ref[...] * _dinv(d0_ref, d1_ref)
    ga_ref[...] = g[:, :DH]
    gb_ref[...] = g[:, DH:]


def _tc2_body(sa_ref, sb_ref, ga_ref, gb_ref, d0_ref, d1_ref, w_ref,
              g2a_ref, g2b_ref):
    dinv = _dinv(d0_ref, d1_ref)
    xa = jnp.maximum(dinv * (sa_ref[...] + ga_ref[...]), 0.0)
    xb = jnp.maximum(dinv * (sb_ref[...] + gb_ref[...]), 0.0)
    x2 = jnp.concatenate([xa, xb], axis=1)
    t = jnp.dot(x2, w_ref[...],
                preferred_element_type=jnp.float32)
    g2 = t * dinv
    g2a_ref[...] = g2[:, :DH]
    g2b_ref[...] = g2[:, DH:]


def _tc3_body(sa_ref, sb_ref, ga_ref, gb_ref, d0_ref, d1_ref, out_ref):
    dinv = _dinv(d0_ref, d1_ref)
    oa = jnp.maximum(dinv * (sa_ref[...] + ga_ref[...]), 0.0)
    ob = jnp.maximum(dinv * (sb_ref[...] + gb_ref[...]), 0.0)
    out_ref[...] = jnp.concatenate([oa, ob], axis=1)


_row_half = pl.BlockSpec((RB, DH), lambda i: (i, 0))
_row_full = pl.BlockSpec((RB, D), lambda i: (i, 0))
_col_deg = pl.BlockSpec((RB, 1), lambda i: (i, 0))
_w_full = pl.BlockSpec((D, D), lambda i: (0, 0))
_half_out = jax.ShapeDtypeStruct((N, DH), jnp.float32)

_tc1a = pl.pallas_call(
    _tc1a_body,
    grid=(N // RB,),
    in_specs=[_row_full, _w_full],
    out_specs=_row_full,
    out_shape=jax.ShapeDtypeStruct((N, D), jnp.float32),
)

_tc1b = pl.pallas_call(
    _tc1b_body,
    grid=(N // RB,),
    in_specs=[_row_full, _col_deg, _col_deg],
    out_specs=[_row_half, _row_half],
    out_shape=[_half_out, _half_out],
)

_tc2 = pl.pallas_call(
    _tc2_body,
    grid=(N // RB,),
    in_specs=[_row_half, _row_half, _row_half, _row_half,
              _col_deg, _col_deg, _w_full],
    out_specs=[_row_half, _row_half],
    out_shape=[_half_out, _half_out],
)

_tc3 = pl.pallas_call(
    _tc3_body,
    grid=(N // RB,),
    in_specs=[_row_half, _row_half, _row_half, _row_half,
              _col_deg, _col_deg],
    out_specs=_row_full,
    out_shape=jax.ShapeDtypeStruct((N, D), jnp.float32),
)


def kernel(x, edge_index, W1, W2):
    src = edge_index[0].astype(jnp.int32)
    dst = edge_index[1].astype(jnp.int32)
    dst2d = dst.reshape(ROWS, K)
    src3d = src.reshape(NS * NPHASE, PH, KS)
    dst3d = dst.reshape(NS * NPHASE, PH, KS)

    h1 = _tc1a(x, W1)
    deg0, deg1 = _sc_hist(dst2d)
    d0 = deg0.reshape(N, 1)
    d1 = deg1.reshape(N, 1)

    ga, gb = _tc1b(h1, d0, d1)
    sa, sb = _sc_scatter(ga, gb, src3d, dst3d)
    g2a, g2b = _tc2(sa, sb, ga, gb, d0, d1, W2)
    s2a, s2b = _sc_scatter(g2a, g2b, src3d, dst3d)
    return _tc3(s2a, s2b, g2a, g2b, d0, d1)
